# baseline (speedup 1.0000x reference)
import jax
import jax.numpy as jnp
from jax.experimental import pallas as pl
from jax.experimental.pallas import tpu as pltpu

HIDDEN = 4096
NUM_EXPERTS = 64
NUM_TOKENS = 32768

BT = 512
K = 4
HC = HIDDEN // K


def _router_block(*refs):
    x_refs = refs[:K]
    wt_refs = refs[K:2 * K]
    b_ref = refs[2 * K]
    o_ref = refs[2 * K + 1]
    acc = jnp.dot(x_refs[0][...], wt_refs[0][...],
                  preferred_element_type=jnp.float32)
    for k in range(1, K):
        acc += jnp.dot(x_refs[k][...], wt_refs[k][...],
                       preferred_element_type=jnp.float32)
    o_ref[...] = acc + b_ref[...]


def kernel(x, W, b):
    wt = W.T
    b2 = b.reshape(1, NUM_EXPERTS)
    grid = (NUM_TOKENS // BT,)
    x_specs = [
        pl.BlockSpec((BT, HC), lambda i, k=k: (i, k)) for k in range(K)
    ]
    wt_specs = [
        pl.BlockSpec((HC, NUM_EXPERTS), lambda i, k=k: (k, 0)) for k in range(K)
    ]
    return pl.pallas_call(
        _router_block,
        grid=grid,
        in_specs=x_specs + wt_specs + [
            pl.BlockSpec((1, NUM_EXPERTS), lambda i: (0, 0)),
        ],
        out_specs=pl.BlockSpec((BT, NUM_EXPERTS), lambda i: (i, 0)),
        out_shape=jax.ShapeDtypeStruct((NUM_TOKENS, NUM_EXPERTS), jnp.float32),
        compiler_params=pltpu.CompilerParams(
            dimension_semantics=("parallel",),
        ),
    )(*([x] * K + [wt] * K + [b2]))


# speedup vs baseline: 1.0208x; 1.0208x over previous
import jax
import jax.numpy as jnp
from jax import lax
from jax.experimental import pallas as pl
from jax.experimental.pallas import tpu as pltpu

HIDDEN = 4096
NUM_EXPERTS = 64
NUM_TOKENS = 32768

BT = 512
K = 4
HC = HIDDEN // K

_DN = (((1,), (1,)), ((), ()))


def _router_block(*refs):
    x_refs = refs[:K]
    w_refs = refs[K:2 * K]
    b_ref = refs[2 * K]
    o_ref = refs[2 * K + 1]
    acc = lax.dot_general(x_refs[0][...], w_refs[0][...], _DN,
                          preferred_element_type=jnp.float32)
    for k in range(1, K):
        acc += lax.dot_general(x_refs[k][...], w_refs[k][...], _DN,
                               preferred_element_type=jnp.float32)
    o_ref[...] = acc + b_ref[...]


def kernel(x, W, b):
    b2 = b.reshape(1, NUM_EXPERTS)
    grid = (NUM_TOKENS // BT,)
    x_specs = [
        pl.BlockSpec((BT, HC), lambda i, k=k: (i, k)) for k in range(K)
    ]
    w_specs = [
        pl.BlockSpec((NUM_EXPERTS, HC), lambda i, k=k: (0, k)) for k in range(K)
    ]
    return pl.pallas_call(
        _router_block,
        grid=grid,
        in_specs=x_specs + w_specs + [
            pl.BlockSpec((1, NUM_EXPERTS), lambda i: (0, 0)),
        ],
        out_specs=pl.BlockSpec((BT, NUM_EXPERTS), lambda i: (i, 0)),
        out_shape=jax.ShapeDtypeStruct((NUM_TOKENS, NUM_EXPERTS), jnp.float32),
        compiler_params=pltpu.CompilerParams(
            dimension_semantics=("parallel",),
        ),
    )(*([x] * K + [W] * K + [b2]))
